# baseline (speedup 1.0000x reference)
import functools

import jax
import jax.numpy as jnp
from jax import lax
from jax.experimental import pallas as pl
from jax.experimental.pallas import tpu as pltpu
from jax.experimental.pallas import tpu_sc as plsc

_K = 8
_L = 16
_N = 8192
_B = 32
_R = 32
_NW = 32
_RPW = (_B * _R) // _NW
_GSZ = _L * _L
_NG = _N // _GSZ
_NEG = float("-inf")


_NBUF = 8


def _sc_body(x_hbm, out_hbm, buf, bm, cand, ebuf, outb,
             sem0, sem1, sem2, sem3, sem4, sem5, sem6, sem7):
    w = lax.axis_index("s") * 2 + lax.axis_index("c")
    row0 = w * _RPW
    iota = lax.iota(jnp.int32, _L)
    neg = jnp.full((_L,), _NEG, jnp.float32)

    sems = (sem0, sem1, sem2, sem3, sem4, sem5, sem6, sem7)

    for p in range(_NBUF - 1):
        pltpu.async_copy(x_hbm.at[row0 + p], buf.at[pl.ds(p * _N, _N)],
                         sems[p])

    def row_step(r, carry):
        par = lax.rem(r, _NBUF)

        @pl.when(r < _RPW - (_NBUF - 1))
        def _start_next():
            src = x_hbm.at[row0 + r + (_NBUF - 1)]
            npar = lax.rem(r + (_NBUF - 1), _NBUF)
            for p in range(_NBUF):
                @pl.when(npar == p)
                def _(p=p):
                    pltpu.async_copy(src, buf.at[pl.ds(p * _N, _N)],
                                     sems[p])

        for p in range(_NBUF):
            @pl.when(par == p)
            def _(p=p):
                pltpu.make_async_copy(x_hbm.at[row0],
                                      buf.at[pl.ds(p * _N, _N)],
                                      sems[p]).wait()

        rb = par * _N

        def a_body(g2, acc):
            for h in range(2):
                g = g2 * 2 + h
                base = rb + g * _GSZ
                m = buf[pl.ds(base, _L)]
                for j in range(1, _L):
                    m = jnp.maximum(m, buf[pl.ds(base + j * _L, _L)])
                bm[pl.ds(g * _L, _L)] = m
                acc = jnp.maximum(acc, m)
            return acc

        lmax = lax.fori_loop(0, _NG // 2, a_body, neg)

        lsort, _ = plsc.sort_key_val(lmax, lmax, descending=True)
        thresh = lsort[_K - 1]

        cnt = jnp.int32(0)
        for g in range(_NG):
            m = bm[pl.ds(g * _L, _L)]
            msk = m >= thresh
            ids = iota + g * _L
            plsc.store_compressed(cand.at[pl.ds(cnt, _L)], ids, mask=msk)
            pc = plsc.all_reduce_population_count(msk)
            cnt = cnt + pc[0]

        def d_cond(st):
            return st[0] * _L < cnt

        def d_body(st):
            c, ecnt = st
            off = c * _L
            lanes_ok = (iota + off) < cnt
            ids = jnp.where(lanes_ok, cand[pl.ds(off, _L)], 0)
            bvec = rb + jnp.right_shift(ids, 4) * _GSZ + \
                jnp.bitwise_and(ids, _L - 1)
            for j in range(_L):
                v = plsc.load_gather(buf, [bvec + j * _L], mask=lanes_ok)
                v = jnp.where(lanes_ok, v, neg)
                emsk = v >= thresh
                plsc.store_compressed(ebuf.at[pl.ds(ecnt, _L)], v,
                                      mask=emsk)
                epc = plsc.all_reduce_population_count(emsk)
                ecnt = ecnt + epc[0]
            return (c + 1, ecnt)

        _, ecnt = lax.while_loop(d_cond, d_body,
                                 (jnp.int32(0), jnp.int32(0)))

        def msort(v):
            s, _ = plsc.sort_key_val(v, v, descending=True)
            return s

        def e_cond(st):
            return st[0] * _L < ecnt

        def e_body(st):
            c, acc = st
            off = c * _L
            v = ebuf[pl.ds(off, _L)]
            v = jnp.where((iota + off) < ecnt, v, neg)
            sv = msort(v)
            acc = msort(jnp.maximum(acc, lax.rev(sv, (0,))))
            return (c + 1, acc)

        _, top = lax.while_loop(e_cond, e_body, (jnp.int32(0), neg))
        plsc.store_compressed(outb.at[pl.ds(r * _K, _L)], top,
                              mask=iota < _K)
        return carry

    lax.fori_loop(0, _RPW, row_step, 0)
    pltpu.sync_copy(outb.at[pl.ds(0, _RPW * _K)], out_hbm.at[w])


def kernel(x):
    b, r, n = x.shape
    xf = x.reshape(b * r, n)
    mesh = plsc.VectorSubcoreMesh(core_axis_name="c", subcore_axis_name="s",
                                  num_cores=2, num_subcores=16)
    run = pl.kernel(
        _sc_body,
        out_type=jax.ShapeDtypeStruct((_B, _R * _K), jnp.float32),
        mesh=mesh,
        scratch_types=[
            pltpu.VMEM((_NBUF * _N,), jnp.float32),
            pltpu.VMEM((_NG * _L,), jnp.float32),
            pltpu.VMEM((_NG * _L + 2 * _L,), jnp.int32),
            pltpu.VMEM((_N + 2 * _L,), jnp.float32),
            pltpu.VMEM((_RPW * _K + _L,), jnp.float32),
            pltpu.SemaphoreType.DMA,
            pltpu.SemaphoreType.DMA,
            pltpu.SemaphoreType.DMA,
            pltpu.SemaphoreType.DMA,
            pltpu.SemaphoreType.DMA,
            pltpu.SemaphoreType.DMA,
            pltpu.SemaphoreType.DMA,
            pltpu.SemaphoreType.DMA,
        ],
        compiler_params=pltpu.CompilerParams(needs_layout_passes=False),
    )
    return run(xf)


# speedup vs baseline: 1.0123x; 1.0123x over previous
import functools

import jax
import jax.numpy as jnp
from jax import lax
from jax.experimental import pallas as pl
from jax.experimental.pallas import tpu as pltpu
from jax.experimental.pallas import tpu_sc as plsc

_K = 8
_L = 16
_N = 8192
_B = 32
_R = 32
_NW = 32
_RPW = (_B * _R) // _NW
_GSZ = _L * _L
_NG = _N // _GSZ
_NEG = float("-inf")


_NBUF = 8


def _sc_body(x_hbm, out_hbm, buf, bm, cand, ebuf, outb,
             sem0, sem1, sem2, sem3, sem4, sem5, sem6, sem7):
    w = lax.axis_index("s") * 2 + lax.axis_index("c")
    row0 = w * _RPW
    iota = lax.iota(jnp.int32, _L)
    neg = jnp.full((_L,), _NEG, jnp.float32)

    sems = (sem0, sem1, sem2, sem3, sem4, sem5, sem6, sem7)

    for p in range(_NBUF - 1):
        pltpu.async_copy(x_hbm.at[row0 + p], buf.at[pl.ds(p * _N, _N)],
                         sems[p])

    def row_step(r, carry):
        par = lax.rem(r, _NBUF)

        @pl.when(r < _RPW - (_NBUF - 1))
        def _start_next():
            src = x_hbm.at[row0 + r + (_NBUF - 1)]
            npar = lax.rem(r + (_NBUF - 1), _NBUF)
            for p in range(_NBUF):
                @pl.when(npar == p)
                def _(p=p):
                    pltpu.async_copy(src, buf.at[pl.ds(p * _N, _N)],
                                     sems[p])

        for p in range(_NBUF):
            @pl.when(par == p)
            def _(p=p):
                pltpu.make_async_copy(x_hbm.at[row0],
                                      buf.at[pl.ds(p * _N, _N)],
                                      sems[p]).wait()

        rb = par * _N

        def a_body(g, acc):
            base = rb + g * _GSZ
            m = buf[pl.ds(base, _L)]
            for j in range(1, _L):
                m = jnp.maximum(m, buf[pl.ds(base + j * _L, _L)])
            bm[pl.ds(g * _L, _L)] = m
            return jnp.maximum(acc, m)

        lmax = lax.fori_loop(0, _NG, a_body, neg)

        lsort, _ = plsc.sort_key_val(lmax, lmax, descending=True)
        thresh = lsort[_K - 1]

        cnt = jnp.int32(0)
        for g in range(_NG):
            m = bm[pl.ds(g * _L, _L)]
            msk = m >= thresh
            ids = iota + g * _L
            plsc.store_compressed(cand.at[pl.ds(cnt, _L)], ids, mask=msk)
            pc = plsc.all_reduce_population_count(msk)
            cnt = cnt + pc[0]

        def d_cond(st):
            return st[0] * _L < cnt

        def d_body(st):
            c, ecnt = st
            off = c * _L
            lanes_ok = (iota + off) < cnt
            ids = jnp.where(lanes_ok, cand[pl.ds(off, _L)], 0)
            bvec = rb + jnp.right_shift(ids, 4) * _GSZ + \
                jnp.bitwise_and(ids, _L - 1)
            for j in range(_L):
                v = plsc.load_gather(buf, [bvec + j * _L], mask=lanes_ok)
                v = jnp.where(lanes_ok, v, neg)
                emsk = v >= thresh
                plsc.store_compressed(ebuf.at[pl.ds(ecnt, _L)], v,
                                      mask=emsk)
                epc = plsc.all_reduce_population_count(emsk)
                ecnt = ecnt + epc[0]
            return (c + 1, ecnt)

        _, ecnt = lax.while_loop(d_cond, d_body,
                                 (jnp.int32(0), jnp.int32(0)))

        def msort(v):
            s, _ = plsc.sort_key_val(v, v, descending=True)
            return s

        def e_cond(st):
            return st[0] * _L < ecnt

        def e_body(st):
            c, acc = st
            off = c * _L
            v = ebuf[pl.ds(off, _L)]
            v = jnp.where((iota + off) < ecnt, v, neg)
            sv = msort(v)
            acc = msort(jnp.maximum(acc, lax.rev(sv, (0,))))
            return (c + 1, acc)

        _, top = lax.while_loop(e_cond, e_body, (jnp.int32(0), neg))
        plsc.store_compressed(outb.at[pl.ds(r * _K, _L)], top,
                              mask=iota < _K)
        return carry

    lax.fori_loop(0, _RPW, row_step, 0)
    pltpu.sync_copy(outb.at[pl.ds(0, _RPW * _K)], out_hbm.at[w])


def kernel(x):
    b, r, n = x.shape
    xf = x.reshape(b * r, n)
    mesh = plsc.VectorSubcoreMesh(core_axis_name="c", subcore_axis_name="s",
                                  num_cores=2, num_subcores=16)
    run = pl.kernel(
        _sc_body,
        out_type=jax.ShapeDtypeStruct((_B, _R * _K), jnp.float32),
        mesh=mesh,
        scratch_types=[
            pltpu.VMEM((_NBUF * _N,), jnp.float32),
            pltpu.VMEM((_NG * _L,), jnp.float32),
            pltpu.VMEM((_NG * _L + 2 * _L,), jnp.int32),
            pltpu.VMEM((_N + 2 * _L,), jnp.float32),
            pltpu.VMEM((_RPW * _K + _L,), jnp.float32),
            pltpu.SemaphoreType.DMA,
            pltpu.SemaphoreType.DMA,
            pltpu.SemaphoreType.DMA,
            pltpu.SemaphoreType.DMA,
            pltpu.SemaphoreType.DMA,
            pltpu.SemaphoreType.DMA,
            pltpu.SemaphoreType.DMA,
            pltpu.SemaphoreType.DMA,
        ],
        compiler_params=pltpu.CompilerParams(needs_layout_passes=False),
    )
    return run(xf)
